# baseline (speedup 1.0000x reference)
import jax
import jax.numpy as jnp
from jax import lax
from jax.experimental import pallas as pl
from jax.experimental.pallas import tpu as pltpu
from jax.experimental.pallas import tpu_sc as plsc

VOCAB = 32000
PAD = 0
SMOOTH = 0.1
CONF = 1.0 - SMOOTH
EPS = SMOOTH / (VOCAB - 2)

N = 2048
ROWS = 128
VS = 25856
SCW = VOCAB - VS
CW = 768
NCH = SCW // CW

NC = 2
NS = 16
NW = NC * NS
PER_W = N // NW
L = 16


def _sc_body(pred_hbm, tgt_hbm, m2_hbm, s2_hbm, sx2_hbm, tv2_hbm,
             buf, macc, sacc, sxacc, tgt_v, fin_v, tv_v, gbuf, dsem, gsem):
    wid = lax.axis_index("s") * NC + lax.axis_index("c")
    base = pl.multiple_of(wid * PER_W, PER_W)
    iota = lax.iota(jnp.int32, L)

    pltpu.sync_copy(tgt_hbm.at[pl.ds(base, PER_W)], tgt_v)

    for kb in range(PER_W // L):
        descs = []
        for j in range(L):
            k = kb * L + j
            tk = tgt_v[pl.ds(kb * L, L)][j]
            c0 = pl.multiple_of(lax.bitwise_and(tk, -128), 128)
            r0 = pl.multiple_of(base + (k & ~7), 8)
            descs.append(pltpu.async_copy(
                pred_hbm.at[pl.ds(r0, 8), pl.ds(c0, 128)], gbuf.at[j],
                gsem))
        tvrow = jnp.zeros((L,), jnp.float32)
        for j in range(L):
            descs[j].wait()
            k = kb * L + j
            tk = tgt_v[pl.ds(kb * L, L)][j]
            lane = lax.bitwise_and(tk, 127)
            acc = jnp.zeros((L,), jnp.float32)
            for v8 in range(8):
                x = gbuf[j, k & 7, pl.ds(v8 * L, L)]
                acc = acc + jnp.where((iota + v8 * L) == lane, x, 0.0)
            tvrow = jnp.where(iota == j, jnp.sum(acc), tvrow)
        tv_v[pl.ds(kb * L, L)] = tvrow
    pltpu.sync_copy(tv_v, tv2_hbm.at[pl.ds(base, PER_W)])

    neg = jnp.full((L,), -1e30, jnp.float32)
    zero = jnp.zeros((L,), jnp.float32)
    for g in range(PER_W // L):
        macc[pl.ds(g * L, L)] = neg
        sacc[pl.ds(g * L, L)] = zero
        sxacc[pl.ds(g * L, L)] = zero

    def chunk_compute(b):
        def do_row(r, _):
            mv = macc[pl.ds(r * L, L)]
            sv = sacc[pl.ds(r * L, L)]
            xv = sxacc[pl.ds(r * L, L)]
            m_c = mv
            sx_c = xv
            for v in range(CW // L):
                x = buf[b, r, pl.ds(v * L, L)]
                m_c = jnp.maximum(m_c, x)
                sx_c = sx_c + x
            s_c = sv * jnp.exp(mv - m_c)
            for v in range(CW // L):
                x = buf[b, r, pl.ds(v * L, L)]
                s_c = s_c + jnp.exp(x - m_c)
            macc[pl.ds(r * L, L)] = m_c
            sacc[pl.ds(r * L, L)] = s_c
            sxacc[pl.ds(r * L, L)] = sx_c
            return 0
        lax.fori_loop(0, PER_W, do_row, 0)

    def start(c, b):
        return pltpu.async_copy(
            pred_hbm.at[pl.ds(base, PER_W), pl.ds(VS + c * CW, CW)],
            buf.at[b], dsem)

    pending = start(0, 0)
    for c in range(NCH):
        pending.wait()
        if c + 1 < NCH:
            pending = start(c + 1, (c + 1) % 2)
        chunk_compute(c % 2)

    for g in range(PER_W // L):
        mrow = jnp.zeros((L,), jnp.float32)
        srow = jnp.zeros((L,), jnp.float32)
        xrow = jnp.zeros((L,), jnp.float32)
        for k in range(L):
            r = g * L + k
            mv = macc[pl.ds(r * L, L)]
            sv = sacc[pl.ds(r * L, L)]
            xv = sxacc[pl.ds(r * L, L)]
            m = jnp.max(mv)
            s = jnp.sum(sv * jnp.exp(mv - m))
            sx = jnp.sum(xv)
            sel = iota == k
            mrow = jnp.where(sel, m, mrow)
            srow = jnp.where(sel, s, srow)
            xrow = jnp.where(sel, sx, xrow)
        fin_v[pl.ds(0 * L, L)] = mrow
        fin_v[pl.ds(1 * L, L)] = srow
        fin_v[pl.ds(2 * L, L)] = xrow
        gb = pl.multiple_of(base + g * L, L)
        pltpu.sync_copy(fin_v.at[pl.ds(0 * L, L)], m2_hbm.at[pl.ds(gb, L)])
        pltpu.sync_copy(fin_v.at[pl.ds(1 * L, L)], s2_hbm.at[pl.ds(gb, L)])
        pltpu.sync_copy(fin_v.at[pl.ds(2 * L, L)], sx2_hbm.at[pl.ds(gb, L)])


def _sc_call(pred2, tgt1):
    mesh = plsc.VectorSubcoreMesh(core_axis_name="c", subcore_axis_name="s")
    f = pl.kernel(
        _sc_body,
        out_type=(
            jax.ShapeDtypeStruct((N,), jnp.float32),
            jax.ShapeDtypeStruct((N,), jnp.float32),
            jax.ShapeDtypeStruct((N,), jnp.float32),
            jax.ShapeDtypeStruct((N,), jnp.float32),
        ),
        mesh=mesh,
        scratch_types=[
            pltpu.VMEM((2, PER_W, CW), jnp.float32),
            pltpu.VMEM((PER_W * L,), jnp.float32),
            pltpu.VMEM((PER_W * L,), jnp.float32),
            pltpu.VMEM((PER_W * L,), jnp.float32),
            pltpu.VMEM((PER_W,), jnp.int32),
            pltpu.VMEM((3 * L,), jnp.float32),
            pltpu.VMEM((PER_W,), jnp.float32),
            pltpu.VMEM((L, 8, 128), jnp.float32),
            pltpu.SemaphoreType.DMA,
            pltpu.SemaphoreType.DMA,
        ],
        compiler_params=pltpu.CompilerParams(needs_layout_passes=False),
    )
    return f(pred2, tgt1)


def _tc_a_body(x_ref, m_ref, s_ref, sx_ref, x0_ref):
    x = x_ref[...]
    m = jnp.max(x, axis=1, keepdims=True)
    m_ref[...] = m
    s_ref[...] = jnp.sum(jnp.exp(x - m), axis=1, keepdims=True)
    sx_ref[...] = jnp.sum(x, axis=1, keepdims=True)
    x0_ref[...] = x[:, 0:1]


def _tc_a(pred2):
    ni = N // ROWS
    o = jax.ShapeDtypeStruct((N, 1), jnp.float32)
    return pl.pallas_call(
        _tc_a_body,
        grid=(ni,),
        in_specs=[pl.BlockSpec((ROWS, VS), lambda i: (i, 0))],
        out_specs=[pl.BlockSpec((ROWS, 1), lambda i: (i, 0))] * 4,
        out_shape=[o, o, o, o],
        compiler_params=pltpu.CompilerParams(
            dimension_semantics=("arbitrary",)),
    )(pred2)


def _tc_b_body(tgt_ref, m1_ref, s1_ref, sx1_ref, x0_ref,
               m2_ref, s2_ref, sx2_ref, tv_ref, out_ref):
    m1 = m1_ref[...]
    m2 = m2_ref[...]
    m = jnp.maximum(m1, m2)
    s = s1_ref[...] * jnp.exp(m1 - m) + s2_ref[...] * jnp.exp(m2 - m)
    lse = m + jnp.log(s)
    sumx = sx1_ref[...] + sx2_ref[...]
    loss = lse - EPS * (sumx - x0_ref[...]) - (CONF - EPS) * tv_ref[...]
    loss = jnp.where(tgt_ref[...] != PAD, loss, 0.0)
    out_ref[0, 0] = jnp.sum(loss)


def _tc_b(*args):
    spec = pl.BlockSpec((N, 1), lambda: (0, 0))
    return pl.pallas_call(
        _tc_b_body,
        in_specs=[spec] * 9,
        out_specs=pl.BlockSpec((1, 1), lambda: (0, 0),
                               memory_space=pltpu.SMEM),
        out_shape=jax.ShapeDtypeStruct((1, 1), jnp.float32),
    )(*args)


def kernel(pred, target):
    pred2 = pred.reshape(N, VOCAB)
    tgt1 = target.astype(jnp.int32).reshape(N)
    tgt2d = tgt1.reshape(N, 1)

    m2, s2, sx2, tv = _sc_call(pred2, tgt1)
    m1, s1, sx1, x0 = _tc_a(pred2)

    out = _tc_b(tgt2d, m1, s1, sx1, x0,
                m2.reshape(N, 1), s2.reshape(N, 1), sx2.reshape(N, 1),
                tv.reshape(N, 1))
    return out[0, 0] / N


# speedup vs baseline: 1.0622x; 1.0622x over previous
import jax
import jax.numpy as jnp
from jax import lax
from jax.experimental import pallas as pl
from jax.experimental.pallas import tpu as pltpu
from jax.experimental.pallas import tpu_sc as plsc

VOCAB = 32000
PAD = 0
SMOOTH = 0.1
CONF = 1.0 - SMOOTH
EPS = SMOOTH / (VOCAB - 2)

N = 2048
ROWS = 128
VS = 24320
SCW = VOCAB - VS
CW = 768
NCH = SCW // CW

NC = 2
NS = 16
NW = NC * NS
PER_W = N // NW
L = 16


def _sc_body(pred_hbm, tgt_hbm, m2_hbm, s2_hbm, ps2_hbm,
             buf, macc, sacc, sxacc, tgt_v, fin_v, tv_v, gbuf, dsem, gsem):
    wid = lax.axis_index("s") * NC + lax.axis_index("c")
    base = pl.multiple_of(wid * PER_W, PER_W)
    iota = lax.iota(jnp.int32, L)

    pltpu.sync_copy(tgt_hbm.at[pl.ds(base, PER_W)], tgt_v)

    for kb in range(PER_W // L):
        descs = []
        for j in range(L):
            k = kb * L + j
            tk = tgt_v[pl.ds(kb * L, L)][j]
            c0 = pl.multiple_of(lax.bitwise_and(tk, -128), 128)
            r0 = pl.multiple_of(base + (k & ~7), 8)
            descs.append(pltpu.async_copy(
                pred_hbm.at[pl.ds(r0, 8), pl.ds(c0, 128)], gbuf.at[j],
                gsem))
        tvrow = jnp.zeros((L,), jnp.float32)
        for j in range(L):
            descs[j].wait()
            k = kb * L + j
            tk = tgt_v[pl.ds(kb * L, L)][j]
            lane = lax.bitwise_and(tk, 127)
            acc = jnp.zeros((L,), jnp.float32)
            for v8 in range(8):
                x = gbuf[j, k & 7, pl.ds(v8 * L, L)]
                acc = acc + jnp.where((iota + v8 * L) == lane, x, 0.0)
            tvrow = jnp.where(iota == j, jnp.sum(acc), tvrow)
        tv_v[pl.ds(kb * L, L)] = tvrow

    neg = jnp.full((L,), -1e30, jnp.float32)
    zero = jnp.zeros((L,), jnp.float32)
    for g in range(PER_W // L):
        macc[pl.ds(g * L, L)] = neg
        sacc[pl.ds(g * L, L)] = zero
        sxacc[pl.ds(g * L, L)] = zero

    def chunk_compute(b):
        def do_row(r, _):
            mv = macc[pl.ds(r * L, L)]
            sv = sacc[pl.ds(r * L, L)]
            xv = sxacc[pl.ds(r * L, L)]
            m_c = mv
            sx_c = xv
            for v in range(CW // L):
                x = buf[b, r, pl.ds(v * L, L)]
                m_c = jnp.maximum(m_c, x)
                sx_c = sx_c + x
            s_c = sv * jnp.exp(mv - m_c)
            for v in range(CW // L):
                x = buf[b, r, pl.ds(v * L, L)]
                s_c = s_c + jnp.exp(x - m_c)
            macc[pl.ds(r * L, L)] = m_c
            sacc[pl.ds(r * L, L)] = s_c
            sxacc[pl.ds(r * L, L)] = sx_c
            return 0
        lax.fori_loop(0, PER_W, do_row, 0)

    def start(c, b):
        return pltpu.async_copy(
            pred_hbm.at[pl.ds(base, PER_W), pl.ds(VS + c * CW, CW)],
            buf.at[b], dsem)

    pending = start(0, 0)
    for c in range(NCH):
        pending.wait()
        if c + 1 < NCH:
            pending = start(c + 1, (c + 1) % 2)
        chunk_compute(c % 2)

    for g in range(PER_W // L):
        mrow = jnp.zeros((L,), jnp.float32)
        srow = jnp.zeros((L,), jnp.float32)
        xrow = jnp.zeros((L,), jnp.float32)
        for k in range(L):
            r = g * L + k
            mv = macc[pl.ds(r * L, L)]
            sv = sacc[pl.ds(r * L, L)]
            xv = sxacc[pl.ds(r * L, L)]
            m = jnp.max(mv)
            s = jnp.sum(sv * jnp.exp(mv - m))
            sx = jnp.sum(xv)
            sel = iota == k
            mrow = jnp.where(sel, m, mrow)
            srow = jnp.where(sel, s, srow)
            xrow = jnp.where(sel, sx, xrow)
        tvrow = tv_v[pl.ds(g * L, L)]
        psrow = -EPS * xrow - (CONF - EPS) * tvrow
        fin_v[pl.ds(0 * L, L)] = mrow
        fin_v[pl.ds(1 * L, L)] = srow
        fin_v[pl.ds(2 * L, L)] = psrow
        gb = pl.multiple_of(base + g * L, L)
        pltpu.sync_copy(fin_v.at[pl.ds(0 * L, L)], m2_hbm.at[pl.ds(gb, L)])
        pltpu.sync_copy(fin_v.at[pl.ds(1 * L, L)], s2_hbm.at[pl.ds(gb, L)])
        pltpu.sync_copy(fin_v.at[pl.ds(2 * L, L)], ps2_hbm.at[pl.ds(gb, L)])


def _sc_call(pred2, tgt1):
    mesh = plsc.VectorSubcoreMesh(core_axis_name="c", subcore_axis_name="s")
    f = pl.kernel(
        _sc_body,
        out_type=(
            jax.ShapeDtypeStruct((N,), jnp.float32),
            jax.ShapeDtypeStruct((N,), jnp.float32),
            jax.ShapeDtypeStruct((N,), jnp.float32),
        ),
        mesh=mesh,
        scratch_types=[
            pltpu.VMEM((2, PER_W, CW), jnp.float32),
            pltpu.VMEM((PER_W * L,), jnp.float32),
            pltpu.VMEM((PER_W * L,), jnp.float32),
            pltpu.VMEM((PER_W * L,), jnp.float32),
            pltpu.VMEM((PER_W,), jnp.int32),
            pltpu.VMEM((3 * L,), jnp.float32),
            pltpu.VMEM((PER_W,), jnp.float32),
            pltpu.VMEM((L, 8, 128), jnp.float32),
            pltpu.SemaphoreType.DMA,
            pltpu.SemaphoreType.DMA,
        ],
        compiler_params=pltpu.CompilerParams(needs_layout_passes=False),
    )
    return f(pred2, tgt1)


def _transpose_col(v):
    rowi = lax.broadcasted_iota(jnp.int32, (ROWS, ROWS), 0)
    coli = lax.broadcasted_iota(jnp.int32, (ROWS, ROWS), 1)
    return jnp.sum(jnp.where(rowi == coli, v, 0.0), axis=0, keepdims=True)


def _tc_a_body(x_ref, m_ref, s_ref, pa_ref):
    x = x_ref[...]
    m = jnp.max(x, axis=1, keepdims=True)
    s = jnp.sum(jnp.exp(x - m), axis=1, keepdims=True)
    sx = jnp.sum(x, axis=1, keepdims=True)
    pa = EPS * (x[:, 0:1] - sx)
    m_ref[...] = _transpose_col(m).reshape(1, 1, ROWS)
    s_ref[...] = _transpose_col(s).reshape(1, 1, ROWS)
    pa_ref[...] = _transpose_col(pa).reshape(1, 1, ROWS)


def _tc_a(pred2):
    ni = N // ROWS
    o = jax.ShapeDtypeStruct((ni, 1, ROWS), jnp.float32)
    return pl.pallas_call(
        _tc_a_body,
        grid=(ni,),
        in_specs=[pl.BlockSpec((ROWS, VS), lambda i: (i, 0))],
        out_specs=[pl.BlockSpec((1, 1, ROWS), lambda i: (i, 0, 0))] * 3,
        out_shape=[o, o, o],
        compiler_params=pltpu.CompilerParams(
            dimension_semantics=("arbitrary",)),
    )(pred2)


def _tc_b_body(tgt_ref, m1_ref, s1_ref, pa_ref, m2_ref, s2_ref, ps_ref,
               out_ref):
    m1 = m1_ref[...]
    m2 = m2_ref[...]
    m = jnp.maximum(m1, m2)
    s = s1_ref[...] * jnp.exp(m1 - m) + s2_ref[...] * jnp.exp(m2 - m)
    lse = m + jnp.log(s)
    loss = lse + pa_ref[...] + ps_ref[...]
    loss = jnp.where(tgt_ref[...] != PAD, loss, 0.0)
    out_ref[0, 0] = jnp.sum(loss) / N


def _tc_b(*args):
    ni = N // ROWS
    spec = pl.BlockSpec((ni, ROWS), lambda: (0, 0))
    return pl.pallas_call(
        _tc_b_body,
        in_specs=[spec] * 7,
        out_specs=pl.BlockSpec((1, 1), lambda: (0, 0),
                               memory_space=pltpu.SMEM),
        out_shape=jax.ShapeDtypeStruct((1, 1), jnp.float32),
    )(*args)


def kernel(pred, target):
    ni = N // ROWS
    pred2 = pred.reshape(N, VOCAB)
    tgt1 = target.astype(jnp.int32).reshape(N)

    m2, s2, ps = _sc_call(pred2, tgt1)
    m1, s1, pa = _tc_a(pred2)

    out = _tc_b(tgt1.reshape(ni, ROWS),
                m1.reshape(ni, ROWS), s1.reshape(ni, ROWS),
                pa.reshape(ni, ROWS),
                m2.reshape(ni, ROWS), s2.reshape(ni, ROWS),
                ps.reshape(ni, ROWS))
    return out[0, 0]


# speedup vs baseline: 1.1527x; 1.0853x over previous
import jax
import jax.numpy as jnp
from jax import lax
from jax.experimental import pallas as pl
from jax.experimental.pallas import tpu as pltpu
from jax.experimental.pallas import tpu_sc as plsc

VOCAB = 32000
PAD = 0
SMOOTH = 0.1
CONF = 1.0 - SMOOTH
EPS = SMOOTH / (VOCAB - 2)

N = 2048
ROWS = 128
VS = 25856
SCW = VOCAB - VS
CW = 768
NCH = SCW // CW

NC = 2
NS = 16
NW = NC * NS
PER_W = N // NW
L = 16


def _sc_body(pred_hbm, tgt_hbm, m2_hbm, s2_hbm, ps2_hbm,
             buf, macc, sacc, sxacc, tgt_v, fin_v, tv_v, gbuf, dsem, gsem):
    wid = lax.axis_index("s") * NC + lax.axis_index("c")
    base = pl.multiple_of(wid * PER_W, PER_W)
    iota = lax.iota(jnp.int32, L)

    first = pltpu.async_copy(
        pred_hbm.at[pl.ds(base, PER_W), pl.ds(VS, CW)], buf.at[0], dsem)

    pltpu.sync_copy(tgt_hbm.at[pl.ds(base, PER_W)], tgt_v)

    for kb in range(PER_W // L):
        descs = []
        for j in range(L):
            k = kb * L + j
            tk = tgt_v[pl.ds(kb * L, L)][j]
            c0 = pl.multiple_of(lax.bitwise_and(tk, -128), 128)
            r0 = pl.multiple_of(base + (k & ~7), 8)
            descs.append(pltpu.async_copy(
                pred_hbm.at[pl.ds(r0, 8), pl.ds(c0, 128)], gbuf.at[j],
                gsem))
        tvrow = jnp.zeros((L,), jnp.float32)
        for j in range(L):
            descs[j].wait()
            k = kb * L + j
            tk = tgt_v[pl.ds(kb * L, L)][j]
            lane = lax.bitwise_and(tk, 127)
            acc = jnp.zeros((L,), jnp.float32)
            for v8 in range(8):
                x = gbuf[j, k & 7, pl.ds(v8 * L, L)]
                acc = acc + jnp.where((iota + v8 * L) == lane, x, 0.0)
            tvrow = jnp.where(iota == j, jnp.sum(acc), tvrow)
        tv_v[pl.ds(kb * L, L)] = tvrow

    neg = jnp.full((L,), -1e30, jnp.float32)
    zero = jnp.zeros((L,), jnp.float32)
    for g in range(PER_W // L):
        macc[pl.ds(g * L, L)] = neg
        sacc[pl.ds(g * L, L)] = zero
        sxacc[pl.ds(g * L, L)] = zero

    def chunk_compute(b):
        def do_row(r, _):
            mv = macc[pl.ds(r * L, L)]
            sv = sacc[pl.ds(r * L, L)]
            xv = sxacc[pl.ds(r * L, L)]
            m_c = mv
            sx_c = xv
            for v in range(CW // L):
                x = buf[b, r, pl.ds(v * L, L)]
                m_c = jnp.maximum(m_c, x)
                sx_c = sx_c + x
            s_c = sv * jnp.exp(mv - m_c)
            for v in range(CW // L):
                x = buf[b, r, pl.ds(v * L, L)]
                s_c = s_c + jnp.exp(x - m_c)
            macc[pl.ds(r * L, L)] = m_c
            sacc[pl.ds(r * L, L)] = s_c
            sxacc[pl.ds(r * L, L)] = sx_c
            return 0
        lax.fori_loop(0, PER_W, do_row, 0)

    def start(c, b):
        return pltpu.async_copy(
            pred_hbm.at[pl.ds(base, PER_W), pl.ds(VS + c * CW, CW)],
            buf.at[b], dsem)

    pending = first
    for c in range(NCH):
        pending.wait()
        if c + 1 < NCH:
            pending = start(c + 1, (c + 1) % 2)
        chunk_compute(c % 2)

    for g in range(PER_W // L):
        mrow = jnp.zeros((L,), jnp.float32)
        srow = jnp.zeros((L,), jnp.float32)
        xrow = jnp.zeros((L,), jnp.float32)
        for k in range(L):
            r = g * L + k
            mv = macc[pl.ds(r * L, L)]
            sv = sacc[pl.ds(r * L, L)]
            xv = sxacc[pl.ds(r * L, L)]
            m = jnp.max(mv)
            s = jnp.sum(sv * jnp.exp(mv - m))
            sx = jnp.sum(xv)
            sel = iota == k
            mrow = jnp.where(sel, m, mrow)
            srow = jnp.where(sel, s, srow)
            xrow = jnp.where(sel, sx, xrow)
        tvrow = tv_v[pl.ds(g * L, L)]
        psrow = -EPS * xrow - (CONF - EPS) * tvrow
        fin_v[pl.ds(0 * L, L)] = mrow
        fin_v[pl.ds(1 * L, L)] = srow
        fin_v[pl.ds(2 * L, L)] = psrow
        gb = pl.multiple_of(base + g * L, L)
        pltpu.sync_copy(fin_v.at[pl.ds(0 * L, L)], m2_hbm.at[pl.ds(gb, L)])
        pltpu.sync_copy(fin_v.at[pl.ds(1 * L, L)], s2_hbm.at[pl.ds(gb, L)])
        pltpu.sync_copy(fin_v.at[pl.ds(2 * L, L)], ps2_hbm.at[pl.ds(gb, L)])


def _sc_call(pred2, tgt1):
    mesh = plsc.VectorSubcoreMesh(core_axis_name="c", subcore_axis_name="s")
    f = pl.kernel(
        _sc_body,
        out_type=(
            jax.ShapeDtypeStruct((N,), jnp.float32),
            jax.ShapeDtypeStruct((N,), jnp.float32),
            jax.ShapeDtypeStruct((N,), jnp.float32),
        ),
        mesh=mesh,
        scratch_types=[
            pltpu.VMEM((2, PER_W, CW), jnp.float32),
            pltpu.VMEM((PER_W * L,), jnp.float32),
            pltpu.VMEM((PER_W * L,), jnp.float32),
            pltpu.VMEM((PER_W * L,), jnp.float32),
            pltpu.VMEM((PER_W,), jnp.int32),
            pltpu.VMEM((3 * L,), jnp.float32),
            pltpu.VMEM((PER_W,), jnp.float32),
            pltpu.VMEM((L, 8, 128), jnp.float32),
            pltpu.SemaphoreType.DMA,
            pltpu.SemaphoreType.DMA,
        ],
        compiler_params=pltpu.CompilerParams(needs_layout_passes=False),
    )
    return f(pred2, tgt1)


def _transpose_col(v):
    rowi = lax.broadcasted_iota(jnp.int32, (ROWS, ROWS), 0)
    coli = lax.broadcasted_iota(jnp.int32, (ROWS, ROWS), 1)
    return jnp.sum(jnp.where(rowi == coli, v, 0.0), axis=0, keepdims=True)


def _tc_a_body(x_ref, m_ref, s_ref, pa_ref):
    x = x_ref[...]
    m = jnp.max(x, axis=1, keepdims=True)
    s = jnp.sum(jnp.exp(x - m), axis=1, keepdims=True)
    sx = jnp.sum(x, axis=1, keepdims=True)
    pa = EPS * (x[:, 0:1] - sx)
    m_ref[...] = _transpose_col(m).reshape(1, 1, ROWS)
    s_ref[...] = _transpose_col(s).reshape(1, 1, ROWS)
    pa_ref[...] = _transpose_col(pa).reshape(1, 1, ROWS)


def _tc_a(pred2):
    ni = N // ROWS
    o = jax.ShapeDtypeStruct((ni, 1, ROWS), jnp.float32)
    return pl.pallas_call(
        _tc_a_body,
        grid=(ni,),
        in_specs=[pl.BlockSpec((ROWS, VS), lambda i: (i, 0))],
        out_specs=[pl.BlockSpec((1, 1, ROWS), lambda i: (i, 0, 0))] * 3,
        out_shape=[o, o, o],
        compiler_params=pltpu.CompilerParams(
            dimension_semantics=("arbitrary",)),
    )(pred2)


def _tc_b_body(tgt_ref, m1_ref, s1_ref, pa_ref, m2_ref, s2_ref, ps_ref,
               out_ref):
    m1 = m1_ref[...]
    m2 = m2_ref[...]
    m = jnp.maximum(m1, m2)
    s = s1_ref[...] * jnp.exp(m1 - m) + s2_ref[...] * jnp.exp(m2 - m)
    lse = m + jnp.log(s)
    loss = lse + pa_ref[...] + ps_ref[...]
    loss = jnp.where(tgt_ref[...] != PAD, loss, 0.0)
    out_ref[0, 0] = jnp.sum(loss) / N


def _tc_b(*args):
    ni = N // ROWS
    spec = pl.BlockSpec((ni, ROWS), lambda: (0, 0))
    return pl.pallas_call(
        _tc_b_body,
        in_specs=[spec] * 7,
        out_specs=pl.BlockSpec((1, 1), lambda: (0, 0),
                               memory_space=pltpu.SMEM),
        out_shape=jax.ShapeDtypeStruct((1, 1), jnp.float32),
    )(*args)


def kernel(pred, target):
    ni = N // ROWS
    pred2 = pred.reshape(N, VOCAB)
    tgt1 = target.astype(jnp.int32).reshape(N)

    m2, s2, ps = _sc_call(pred2, tgt1)
    m1, s1, pa = _tc_a(pred2)

    out = _tc_b(tgt1.reshape(ni, ROWS),
                m1.reshape(ni, ROWS), s1.reshape(ni, ROWS),
                pa.reshape(ni, ROWS),
                m2.reshape(ni, ROWS), s2.reshape(ni, ROWS),
                ps.reshape(ni, ROWS))
    return out[0, 0]
